# Initial kernel scaffold; baseline (speedup 1.0000x reference)
import jax
import jax.numpy as jnp
from jax.experimental import pallas as pl


def kernel(x, input_mask, edge_index, edge_weight, attention, Wz, bz, Lz, lbz, Wr, br, Lr, lbr, Wh, bh, Lh, lbh, W1, b1, W2, b2):
    raise NotImplementedError("write your pallas kernel here")


# speedup vs baseline: 337.9636x; 337.9636x over previous
import functools

import jax
import jax.numpy as jnp
from jax import lax
from jax.experimental import pallas as pl
from jax.experimental.pallas import tpu as pltpu
from jax.experimental.pallas import tpu_sc as plsc

N = 10000
E = 160000
P = 12
B = 16
NPAD = 10112
TILE_N = 632
EPAD = 163840
GW = 128
NG = 3

RL = 640

_SPLAT_DNUMS = lax.GatherDimensionNumbers(
    offset_dims=(), collapsed_slice_dims=(0,), start_index_map=(0,))


def _splat(vec16, lane):
    idx = jnp.full((16,), lane, jnp.int32)
    return lax.gather(vec16, idx[:, None], _SPLAT_DNUMS, (1,),
                      mode=lax.GatherScatterMode.PROMISE_IN_BOUNDS)


@functools.cache
def _sc_mesh():
    return plsc.VectorSubcoreMesh(core_axis_name="c", subcore_axis_name="s")


@functools.cache
def _deg_kernel():
    C = 40

    @functools.partial(
        pl.kernel,
        mesh=_sc_mesh(),
        out_type=jax.ShapeDtypeStruct((2, NPAD, GW), jnp.float32),
        scratch_types=[
            pltpu.VMEM((C, 128), jnp.int32),
            pltpu.VMEM((C * 128,), jnp.float32),
            pltpu.VMEM((128, GW), jnp.float32),
            pltpu.VMEM_SHARED((NPAD, GW), jnp.float32),
        ],
    )
    def deg_kernel(dst_hbm, w_hbm, out_hbm, dst_loc, w_loc, wbuf, deg_sh):
        c = lax.axis_index("c")
        t = lax.axis_index("s")
        pltpu.sync_copy(dst_hbm.at[c, t], dst_loc)
        pltpu.sync_copy(w_hbm.at[c, t], w_loc)

        zf = jnp.zeros((16,), jnp.float32)

        def _zero_wbuf(e, carry):
            for q in range(GW // 16):
                wbuf[e, pl.ds(q * 16, 16)] = zf
            return carry

        lax.fori_loop(0, 128, _zero_wbuf, 0)
        for k in range(4):
            pltpu.sync_copy(wbuf, deg_sh.at[pl.ds(t * TILE_N + k * 128, 128)])
        pltpu.sync_copy(wbuf.at[pl.ds(0, TILE_N - 512)],
                        deg_sh.at[pl.ds(t * TILE_N + 512, TILE_N - 512)])
        plsc.subcore_barrier()

        lane0 = lax.iota(jnp.int32, 16) == 0

        def _chunk(j, carry):
            for g in range(8):
                wv = w_loc[pl.ds(j * 128 + g * 16, 16)]

                def _spread(l, carry2, g=g, wv=wv):
                    row = jnp.where(lane0, _splat(wv, l), 0.0)
                    wbuf[g * 16 + l, pl.ds(0, 16)] = row
                    return carry2

                lax.fori_loop(0, 16, _spread, 0)
            pltpu.sync_copy(wbuf, deg_sh.at[dst_loc.at[j]], add=True)
            return carry

        lax.fori_loop(0, C, _chunk, 0)
        plsc.subcore_barrier()
        pltpu.sync_copy(deg_sh.at[pl.ds(t * TILE_N, TILE_N)],
                        out_hbm.at[c, pl.ds(t * TILE_N, TILE_N)])

    return deg_kernel


@functools.cache
def _spmm_kernel(C):
    @functools.partial(
        pl.kernel,
        mesh=_sc_mesh(),
        out_type=jax.ShapeDtypeStruct((2, NPAD, GW), jnp.float32),
        scratch_types=[
            pltpu.VMEM((C, 128), jnp.int32),
            pltpu.VMEM((C, 128), jnp.int32),
            pltpu.VMEM((C * 128,), jnp.float32),
            pltpu.VMEM((128, GW), jnp.float32),
            pltpu.VMEM_SHARED((NPAD, GW), jnp.float32),
            pltpu.SemaphoreType.DMA,
        ],
    )
    def spmm_kernel(f_hbm, src_hbm, dst_hbm, w_hbm, out_hbm,
                    src_loc, dst_loc, w_loc, rows, agg_sh, sem):
        c = lax.axis_index("c")
        t = lax.axis_index("s")
        pltpu.sync_copy(src_hbm.at[c, t], src_loc)
        pltpu.sync_copy(dst_hbm.at[c, t], dst_loc)
        pltpu.sync_copy(w_hbm.at[c, t], w_loc)

        zf = jnp.zeros((16,), jnp.float32)

        def _zero_rows(e, carry):
            for q in range(GW // 16):
                rows[e, pl.ds(q * 16, 16)] = zf
            return carry

        lax.fori_loop(0, 128, _zero_rows, 0)
        for k in range(4):
            pltpu.sync_copy(rows, agg_sh.at[pl.ds(t * TILE_N + k * 128, 128)])
        pltpu.sync_copy(rows.at[pl.ds(0, TILE_N - 512)],
                        agg_sh.at[pl.ds(t * TILE_N + 512, TILE_N - 512)])
        plsc.subcore_barrier()

        def _chunk(j, carry):
            pltpu.async_copy(f_hbm.at[src_loc.at[j]], rows, sem).wait()
            for g in range(8):
                wv = w_loc[pl.ds(j * 128 + g * 16, 16)]

                def _scale(l, carry2, g=g, wv=wv):
                    sp = _splat(wv, l)
                    e = g * 16 + l
                    for q in range(GW // 16):
                        rows[e, pl.ds(q * 16, 16)] = (
                            rows[e, pl.ds(q * 16, 16)] * sp)
                    return carry2

                lax.fori_loop(0, 16, _scale, 0)
            pltpu.sync_copy(rows, agg_sh.at[dst_loc.at[j]], add=True)
            return carry

        lax.fori_loop(0, C, _chunk, 0)
        plsc.subcore_barrier()
        pltpu.sync_copy(agg_sh.at[pl.ds(t * TILE_N, TILE_N)],
                        out_hbm.at[c, pl.ds(t * TILE_N, TILE_N)])

    return spmm_kernel


def _tc_body(at_ref, xt_ref, mt_ref, dv_ref, ballt_ref, cball_ref,
             lzt_ref, lrt_ref, lht_ref, att_ref, w1t_ref, b1_ref,
             w2t_ref, b2_ref, res_ref, imp_ref):
    at = at_ref[...]
    xt = xt_ref[...]
    mt = mt_ref[...]
    dv = dv_ref[...]
    dsq = dv * dv
    a0 = at[:P] * dv + dsq * xt
    a1 = at[P:] * dv + dsq * mt
    atp = jnp.concatenate([a0, a1], axis=0)
    call = (jnp.dot(ballt_ref[...], atp, preferred_element_type=jnp.float32)
            + cball_ref[...])

    av = att_ref[...]
    ex = jnp.exp(av - jnp.max(av))
    pr = ex / jnp.sum(ex)

    lzt = lzt_ref[...]
    lrt = lrt_ref[...]
    lht = lht_ref[...]

    h = jnp.zeros((P, at.shape[1]), jnp.float32)
    hacc = jnp.zeros((P, at.shape[1]), jnp.float32)
    for p in range(P):
        cz = call[p * P:(p + 1) * P]
        cr = call[P * P + p * P: P * P + (p + 1) * P]
        chh = call[2 * P * P + p * P: 2 * P * P + (p + 1) * P]
        z = jax.nn.sigmoid(cz + jnp.dot(lzt, h, preferred_element_type=jnp.float32))
        r = jax.nn.sigmoid(cr + jnp.dot(lrt, h, preferred_element_type=jnp.float32))
        ht = jnp.tanh(chh + jnp.dot(lht, h * r, preferred_element_type=jnp.float32))
        h = z * h + (1.0 - z) * ht
        hacc = hacc + pr[p:p + 1] * h

    o = jnp.maximum(hacc, 0.0)
    o = jnp.maximum(jnp.dot(w1t_ref[...], o, preferred_element_type=jnp.float32)
                    + b1_ref[...], 0.0)
    o = jax.nn.sigmoid(jnp.dot(w2t_ref[...], o, preferred_element_type=jnp.float32)
                       + b2_ref[...])
    imp_ref[...] = o
    res_ref[...] = mt * xt + (1.0 - mt) * o


def _mk_gate_mat(m):
    eye = jnp.eye(P, dtype=jnp.float32)
    return (eye[:, None, None, :] * m.T[None, :, :, None]).reshape(P * P, 2 * P)


def kernel(x, input_mask, edge_index, edge_weight, attention, Wz, bz, Lz, lbz,
           Wr, br, Lr, lbr, Wh, bh, Lh, lbh, W1, b1, W2, b2):
    src = edge_index[0]
    dst = edge_index[1]
    padn = EPAD - E
    srcp = jnp.concatenate([src, jnp.zeros((padn,), jnp.int32)])
    dstp = jnp.concatenate([dst, jnp.zeros((padn,), jnp.int32)])
    wp = jnp.concatenate([edge_weight, jnp.zeros((padn,), jnp.float32)])

    dst_k1 = dstp.reshape(2, 16, 40, 128)
    w_k1 = wp.reshape(2, 16, 40 * 128)

    degw = _deg_kernel()(dst_k1, w_k1)
    deg = degw[:, :N, 0].sum(axis=0) + 1.0
    dinv = lax.rsqrt(deg)
    dpad = jnp.pad(dinv, (0, NPAD - N))

    noise = jax.random.uniform(jax.random.key(42), x.shape, dtype=x.dtype) * 0.01
    xg = input_mask * x + (1.0 - input_mask) * noise
    stk = jnp.stack([xg, input_mask], axis=2)
    fn = stk.transpose(3, 0, 2, 1).reshape(N, NG, GW)
    fn = jnp.pad(fn, ((0, NPAD - N), (0, 0), (0, 0)))
    f3 = (fn * dpad[:, None, None]).transpose(1, 0, 2).reshape(NG * NPAD, GW)

    sA = srcp.reshape(16, 80, 128)
    srcA = jnp.stack([sA, sA + NPAD], axis=0)
    dstA = jnp.broadcast_to(dstp.reshape(1, 16, 80, 128), (2, 16, 80, 128))
    wA = jnp.broadcast_to(wp.reshape(1, 16, 80 * 128), (2, 16, 80 * 128))
    aggA = _spmm_kernel(80)(f3, srcA, dstA, wA)

    srcB = srcp.reshape(2, 16, 40, 128) + 2 * NPAD
    dstB = dstp.reshape(2, 16, 40, 128)
    wB = wp.reshape(2, 16, 40 * 128)
    aggB = _spmm_kernel(40)(f3, srcB, dstB, wB)

    agg = jnp.concatenate([aggA[0], aggA[1], aggB[0] + aggB[1]],
                          axis=1)

    a = agg[:N].reshape(N, B, 2, P)
    at = a.transpose(2, 3, 0, 1).reshape(2 * P, N * B)
    xgt = xg.transpose(1, 2, 0).reshape(P, N * B)
    mt = input_mask.transpose(1, 2, 0).reshape(P, N * B)
    dvr = jnp.repeat(dinv, B).reshape(1, N * B)

    mz = Wz @ Lz[:P]
    mr = Wr @ Lr[:P]
    mh = Wh @ Lh[:P]
    cz = bz @ Lz[:P] + lbz
    cr = br @ Lr[:P] + lbr
    chh = bh @ Lh[:P] + lbh
    ballt = jnp.concatenate([_mk_gate_mat(mz), _mk_gate_mat(mr),
                             _mk_gate_mat(mh)], axis=0)
    cball = jnp.concatenate([jnp.tile(cz, P), jnp.tile(cr, P),
                             jnp.tile(chh, P)])[:, None]

    grid = (N * B) // RL
    full = lambda shape: pl.BlockSpec(shape, lambda i: (0, 0))
    rest, rimp = pl.pallas_call(
        _tc_body,
        grid=(grid,),
        in_specs=[
            pl.BlockSpec((2 * P, RL), lambda i: (0, i)),
            pl.BlockSpec((P, RL), lambda i: (0, i)),
            pl.BlockSpec((P, RL), lambda i: (0, i)),
            pl.BlockSpec((1, RL), lambda i: (0, i)),
            full((3 * P * P, 2 * P)),
            full((3 * P * P, 1)),
            full((P, P)),
            full((P, P)),
            full((P, P)),
            full((P, 1)),
            full((P, P)),
            full((P, 1)),
            full((P, P)),
            full((P, 1)),
        ],
        out_specs=[
            pl.BlockSpec((P, RL), lambda i: (0, i)),
            pl.BlockSpec((P, RL), lambda i: (0, i)),
        ],
        out_shape=[
            jax.ShapeDtypeStruct((P, N * B), jnp.float32),
            jax.ShapeDtypeStruct((P, N * B), jnp.float32),
        ],
    )(at, xgt, mt, dvr, ballt, cball, Lz[P:].T, Lr[P:].T, Lh[P:].T,
      attention[:, None], W1.T, b1[:, None], W2.T, b2[:, None])

    res = rest.reshape(P, N, B).transpose(2, 0, 1)
    imputation = rimp.reshape(P, N, B).transpose(2, 0, 1)
    return (res, imputation)


# speedup vs baseline: 357.6396x; 1.0582x over previous
import functools

import jax
import jax.numpy as jnp
from jax import lax
from jax.experimental import pallas as pl
from jax.experimental.pallas import tpu as pltpu
from jax.experimental.pallas import tpu_sc as plsc

N = 10000
E = 160000
P = 12
B = 16
NPAD = 10112
TILE_N = 632
EPAD = 163840
GW = 128
NG = 3

RL = 640

_SPLAT_DNUMS = lax.GatherDimensionNumbers(
    offset_dims=(), collapsed_slice_dims=(0,), start_index_map=(0,))


def _splat(vec16, lane):
    idx = jnp.full((16,), lane, jnp.int32)
    return lax.gather(vec16, idx[:, None], _SPLAT_DNUMS, (1,),
                      mode=lax.GatherScatterMode.PROMISE_IN_BOUNDS)


@functools.cache
def _sc_mesh():
    return plsc.VectorSubcoreMesh(core_axis_name="c", subcore_axis_name="s")


@functools.cache
def _deg_kernel():
    C = 40

    @functools.partial(
        pl.kernel,
        mesh=_sc_mesh(),
        out_type=jax.ShapeDtypeStruct((2, NPAD, GW), jnp.float32),
        scratch_types=[
            pltpu.VMEM((C, 128), jnp.int32),
            pltpu.VMEM((C * 128,), jnp.float32),
            pltpu.VMEM((128, GW), jnp.float32),
            pltpu.VMEM_SHARED((NPAD, GW), jnp.float32),
        ],
    )
    def deg_kernel(dst_hbm, w_hbm, out_hbm, dst_loc, w_loc, wbuf, deg_sh):
        c = lax.axis_index("c")
        t = lax.axis_index("s")
        pltpu.sync_copy(dst_hbm.at[c, t], dst_loc)
        pltpu.sync_copy(w_hbm.at[c, t], w_loc)

        zf = jnp.zeros((16,), jnp.float32)

        def _zero_wbuf(e, carry):
            for q in range(GW // 16):
                wbuf[e, pl.ds(q * 16, 16)] = zf
            return carry

        lax.fori_loop(0, 128, _zero_wbuf, 0)
        for k in range(4):
            pltpu.sync_copy(wbuf, deg_sh.at[pl.ds(t * TILE_N + k * 128, 128)])
        pltpu.sync_copy(wbuf.at[pl.ds(0, TILE_N - 512)],
                        deg_sh.at[pl.ds(t * TILE_N + 512, TILE_N - 512)])
        plsc.subcore_barrier()

        lane0 = lax.iota(jnp.int32, 16) == 0

        def _chunk(j, carry):
            for g in range(8):
                wv = w_loc[pl.ds(j * 128 + g * 16, 16)]

                def _spread(l, carry2, g=g, wv=wv):
                    row = jnp.where(lane0, _splat(wv, l), 0.0)
                    wbuf[g * 16 + l, pl.ds(0, 16)] = row
                    return carry2

                lax.fori_loop(0, 16, _spread, 0)
            pltpu.sync_copy(wbuf, deg_sh.at[dst_loc.at[j]], add=True)
            return carry

        lax.fori_loop(0, C, _chunk, 0)
        plsc.subcore_barrier()
        pltpu.sync_copy(deg_sh.at[pl.ds(t * TILE_N, TILE_N)],
                        out_hbm.at[c, pl.ds(t * TILE_N, TILE_N)])

    return deg_kernel


@functools.cache
def _spmm_kernel(C):
    HS = 2 if C >= 80 else 1
    C2 = C // HS

    @functools.partial(
        pl.kernel,
        mesh=_sc_mesh(),
        out_type=jax.ShapeDtypeStruct((2, NPAD, GW), jnp.float32),
        scratch_types=[
            pltpu.VMEM((C2, 128), jnp.int32),
            pltpu.VMEM((C2, 128), jnp.int32),
            pltpu.VMEM((C2 * 128,), jnp.float32),
            pltpu.VMEM((128, GW), jnp.float32),
            pltpu.VMEM((128, GW), jnp.float32),
            pltpu.VMEM_SHARED((NPAD, GW), jnp.float32),
            pltpu.SemaphoreType.DMA,
            pltpu.SemaphoreType.DMA,
        ],
    )
    def spmm_kernel(f_hbm, src_hbm, dst_hbm, w_hbm, out_hbm,
                    src_loc, dst_loc, w_loc, rows0, rows1, agg_sh,
                    sem0, sem1):
        c = lax.axis_index("c")
        t = lax.axis_index("s")

        zf = jnp.zeros((16,), jnp.float32)

        def _zero_rows(e, carry):
            for q in range(GW // 16):
                rows0[e, pl.ds(q * 16, 16)] = zf
            return carry

        lax.fori_loop(0, 128, _zero_rows, 0)
        for k in range(4):
            pltpu.sync_copy(rows0, agg_sh.at[pl.ds(t * TILE_N + k * 128, 128)])
        pltpu.sync_copy(rows0.at[pl.ds(0, TILE_N - 512)],
                        agg_sh.at[pl.ds(t * TILE_N + 512, TILE_N - 512)])
        plsc.subcore_barrier()

        def _scale_rows(buf, j):
            for g in range(8):
                wv = w_loc[pl.ds(j * 128 + g * 16, 16)]

                def _scale4(q4, carry2, g=g, wv=wv):
                    for li in range(4):
                        l = q4 * 4 + li
                        sp = _splat(wv, l)
                        e = g * 16 + l
                        for q in range(GW // 16):
                            buf[e, pl.ds(q * 16, 16)] = (
                                buf[e, pl.ds(q * 16, 16)] * sp)
                    return carry2

                lax.fori_loop(0, 4, _scale4, 0)

        for h in range(HS):
            pltpu.sync_copy(src_hbm.at[c, t, pl.ds(h * C2, C2)], src_loc)
            pltpu.sync_copy(dst_hbm.at[c, t, pl.ds(h * C2, C2)], dst_loc)
            pltpu.sync_copy(w_hbm.at[c, t, pl.ds(h * C2 * 128, C2 * 128)],
                            w_loc)
            pltpu.async_copy(f_hbm.at[src_loc.at[0]], rows0, sem0)

            def _pair(j2, carry):
                j = j2 * 2
                pltpu.async_copy(f_hbm.at[src_loc.at[j + 1]], rows1, sem1)
                pltpu.make_async_copy(f_hbm.at[src_loc.at[j]], rows0,
                                      sem0).wait()
                _scale_rows(rows0, j)
                pltpu.sync_copy(rows0, agg_sh.at[dst_loc.at[j]], add=True)

                jn = jnp.minimum(j + 2, C2 - 1)
                pltpu.async_copy(f_hbm.at[src_loc.at[jn]], rows0, sem0)
                pltpu.make_async_copy(f_hbm.at[src_loc.at[j + 1]], rows1,
                                      sem1).wait()
                _scale_rows(rows1, j + 1)
                pltpu.sync_copy(rows1, agg_sh.at[dst_loc.at[j + 1]], add=True)
                return carry

            lax.fori_loop(0, C2 // 2, _pair, 0)
            pltpu.make_async_copy(f_hbm.at[src_loc.at[C2 - 1]], rows0,
                                  sem0).wait()
        plsc.subcore_barrier()
        pltpu.sync_copy(agg_sh.at[pl.ds(t * TILE_N, TILE_N)],
                        out_hbm.at[c, pl.ds(t * TILE_N, TILE_N)])

    return spmm_kernel


def _tc_body(at_ref, xt_ref, mt_ref, dv_ref, ballt_ref, cball_ref,
             lzt_ref, lrt_ref, lht_ref, att_ref, w1t_ref, b1_ref,
             w2t_ref, b2_ref, res_ref, imp_ref):
    at = at_ref[...]
    xt = xt_ref[...]
    mt = mt_ref[...]
    dv = dv_ref[...]
    dsq = dv * dv
    a0 = at[:P] * dv + dsq * xt
    a1 = at[P:] * dv + dsq * mt
    atp = jnp.concatenate([a0, a1], axis=0)
    call = (jnp.dot(ballt_ref[...], atp, preferred_element_type=jnp.float32)
            + cball_ref[...])

    av = att_ref[...]
    ex = jnp.exp(av - jnp.max(av))
    pr = ex / jnp.sum(ex)

    lzt = lzt_ref[...]
    lrt = lrt_ref[...]
    lht = lht_ref[...]

    h = jnp.zeros((P, at.shape[1]), jnp.float32)
    hacc = jnp.zeros((P, at.shape[1]), jnp.float32)
    for p in range(P):
        cz = call[p * P:(p + 1) * P]
        cr = call[P * P + p * P: P * P + (p + 1) * P]
        chh = call[2 * P * P + p * P: 2 * P * P + (p + 1) * P]
        z = jax.nn.sigmoid(cz + jnp.dot(lzt, h, preferred_element_type=jnp.float32))
        r = jax.nn.sigmoid(cr + jnp.dot(lrt, h, preferred_element_type=jnp.float32))
        ht = jnp.tanh(chh + jnp.dot(lht, h * r, preferred_element_type=jnp.float32))
        h = z * h + (1.0 - z) * ht
        hacc = hacc + pr[p:p + 1] * h

    o = jnp.maximum(hacc, 0.0)
    o = jnp.maximum(jnp.dot(w1t_ref[...], o, preferred_element_type=jnp.float32)
                    + b1_ref[...], 0.0)
    o = jax.nn.sigmoid(jnp.dot(w2t_ref[...], o, preferred_element_type=jnp.float32)
                       + b2_ref[...])
    imp_ref[...] = o
    res_ref[...] = mt * xt + (1.0 - mt) * o


def _mk_gate_mat(m):
    eye = jnp.eye(P, dtype=jnp.float32)
    return (eye[:, None, None, :] * m.T[None, :, :, None]).reshape(P * P, 2 * P)


def kernel(x, input_mask, edge_index, edge_weight, attention, Wz, bz, Lz, lbz,
           Wr, br, Lr, lbr, Wh, bh, Lh, lbh, W1, b1, W2, b2):
    src = edge_index[0]
    dst = edge_index[1]
    padn = EPAD - E
    srcp = jnp.concatenate([src, jnp.zeros((padn,), jnp.int32)])
    dstp = jnp.concatenate([dst, jnp.zeros((padn,), jnp.int32)])
    wp = jnp.concatenate([edge_weight, jnp.zeros((padn,), jnp.float32)])

    dst_k1 = dstp.reshape(2, 16, 40, 128)
    w_k1 = wp.reshape(2, 16, 40 * 128)

    degw = _deg_kernel()(dst_k1, w_k1)
    deg = degw[:, :N, 0].sum(axis=0) + 1.0
    dinv = lax.rsqrt(deg)
    dpad = jnp.pad(dinv, (0, NPAD - N))

    noise = jax.random.uniform(jax.random.key(42), x.shape, dtype=x.dtype) * 0.01
    xg = input_mask * x + (1.0 - input_mask) * noise
    stk = jnp.stack([xg, input_mask], axis=2)
    fn = stk.transpose(3, 0, 2, 1).reshape(N, NG, GW)
    fn = jnp.pad(fn, ((0, NPAD - N), (0, 0), (0, 0)))
    f3 = (fn * dpad[:, None, None]).transpose(1, 0, 2).reshape(NG * NPAD, GW)

    sA = srcp.reshape(16, 80, 128)
    srcA = jnp.stack([sA, sA + NPAD], axis=0)
    dstA = jnp.broadcast_to(dstp.reshape(1, 16, 80, 128), (2, 16, 80, 128))
    wA = jnp.broadcast_to(wp.reshape(1, 16, 80 * 128), (2, 16, 80 * 128))
    aggA = _spmm_kernel(80)(f3, srcA, dstA, wA)

    srcB = srcp.reshape(2, 16, 40, 128) + 2 * NPAD
    dstB = dstp.reshape(2, 16, 40, 128)
    wB = wp.reshape(2, 16, 40 * 128)
    aggB = _spmm_kernel(40)(f3, srcB, dstB, wB)

    agg = jnp.concatenate([aggA[0], aggA[1], aggB[0] + aggB[1]],
                          axis=1)

    a = agg[:N].reshape(N, B, 2, P)
    at = a.transpose(2, 3, 0, 1).reshape(2 * P, N * B)
    xgt = xg.transpose(1, 2, 0).reshape(P, N * B)
    mt = input_mask.transpose(1, 2, 0).reshape(P, N * B)
    dvr = jnp.repeat(dinv, B).reshape(1, N * B)

    mz = Wz @ Lz[:P]
    mr = Wr @ Lr[:P]
    mh = Wh @ Lh[:P]
    cz = bz @ Lz[:P] + lbz
    cr = br @ Lr[:P] + lbr
    chh = bh @ Lh[:P] + lbh
    ballt = jnp.concatenate([_mk_gate_mat(mz), _mk_gate_mat(mr),
                             _mk_gate_mat(mh)], axis=0)
    cball = jnp.concatenate([jnp.tile(cz, P), jnp.tile(cr, P),
                             jnp.tile(chh, P)])[:, None]

    grid = (N * B) // RL
    full = lambda shape: pl.BlockSpec(shape, lambda i: (0, 0))
    rest, rimp = pl.pallas_call(
        _tc_body,
        grid=(grid,),
        in_specs=[
            pl.BlockSpec((2 * P, RL), lambda i: (0, i)),
            pl.BlockSpec((P, RL), lambda i: (0, i)),
            pl.BlockSpec((P, RL), lambda i: (0, i)),
            pl.BlockSpec((1, RL), lambda i: (0, i)),
            full((3 * P * P, 2 * P)),
            full((3 * P * P, 1)),
            full((P, P)),
            full((P, P)),
            full((P, P)),
            full((P, 1)),
            full((P, P)),
            full((P, 1)),
            full((P, P)),
            full((P, 1)),
        ],
        out_specs=[
            pl.BlockSpec((P, RL), lambda i: (0, i)),
            pl.BlockSpec((P, RL), lambda i: (0, i)),
        ],
        out_shape=[
            jax.ShapeDtypeStruct((P, N * B), jnp.float32),
            jax.ShapeDtypeStruct((P, N * B), jnp.float32),
        ],
    )(at, xgt, mt, dvr, ballt, cball, Lz[P:].T, Lr[P:].T, Lh[P:].T,
      attention[:, None], W1.T, b1[:, None], W2.T, b2[:, None])

    res = rest.reshape(P, N, B).transpose(2, 0, 1)
    imputation = rimp.reshape(P, N, B).transpose(2, 0, 1)
    return (res, imputation)
